# baseline (speedup 1.0000x reference)
import functools

import jax
import jax.numpy as jnp
from jax import lax
from jax.experimental import pallas as pl
from jax.experimental.pallas import tpu as pltpu
from jax.experimental.pallas import tpu_sc as plsc

NC = 2
NS = 16
L = 16
NW = NC * NS
NSEG = 64
BLK = 2000
H = BLK // 2


def _sc_kernel(n_edges):
    ept = n_edges // NW
    nblk = ept // BLK
    mesh = plsc.VectorSubcoreMesh(core_axis_name="c", subcore_axis_name="s")

    @functools.partial(
        pl.kernel,
        out_type=jax.ShapeDtypeStruct((NC, NSEG), jnp.float32),
        mesh=mesh,
        scratch_types=[
            [pltpu.VMEM((BLK,), jnp.int32) for _ in range(2)],
            [pltpu.VMEM((BLK,), jnp.int32) for _ in range(2)],
            [pltpu.VMEM((BLK,), jnp.int32) for _ in range(2)],
            pltpu.VMEM((BLK, 4), jnp.float32),
            pltpu.VMEM((BLK, 4), jnp.float32),
            pltpu.VMEM((128,), jnp.float32),
            pltpu.VMEM((NS * NSEG,), jnp.float32),
            pltpu.VMEM((NSEG,), jnp.float32),
            pltpu.VMEM((NS, NSEG), jnp.float32),
            pltpu.VMEM_SHARED((NS, NSEG), jnp.float32),
            [pltpu.SemaphoreType.DMA for _ in range(3)],
            [pltpu.SemaphoreType.DMA for _ in range(4)],
        ],
        compiler_params=pltpu.CompilerParams(
            needs_layout_passes=False, use_tc_tiling_on_sc=False),
    )
    def k(tab, mp, mb, sg, out, idx_i, idx_j, bat, rows_i, rows_j, sig_v,
          acc, part, red, shared, sem_i, sem_g):
        c = lax.axis_index("c")
        s = lax.axis_index("s")
        wid = c * NS + s
        tile_base = wid * ept

        pltpu.sync_copy(sg, sig_v)

        zero = jnp.zeros((L,), jnp.float32)

        def zbody(kk, _):
            acc[pl.ds(kk * L, L)] = zero
            return 0

        lax.fori_loop(0, (NS * NSEG) // L, zbody, 0)

        iota = lax.iota(jnp.int32, L)
        lane_base = iota * NSEG
        c0 = jnp.full((L,), 0, jnp.int32)
        c1 = jnp.full((L,), 1, jnp.int32)
        c2 = jnp.full((L,), 2, jnp.int32)
        c3 = jnp.full((L,), 3, jnp.int32)

        def compute(u, vlo, vhi):
            ri = rows_i
            rj = rows_j
            bt = bat[u]

            def v_body(v, _):
                e = v * L + iota
                xi = plsc.load_gather(ri, [e, c0])
                yi = plsc.load_gather(ri, [e, c1])
                zi = plsc.load_gather(ri, [e, c2])
                ti = plsc.load_gather(ri, [e, c3])
                xj = plsc.load_gather(rj, [e, c0])
                yj = plsc.load_gather(rj, [e, c1])
                zj = plsc.load_gather(rj, [e, c2])
                tj = plsc.load_gather(rj, [e, c3])
                dx = xi - xj
                dy = yi - yj
                dz = zi - zj
                d2 = dx * dx + dy * dy + dz * dz + 1e-12
                d6 = d2 * d2 * d2
                sidx = (ti * 10.0 + tj).astype(jnp.int32)
                s6 = plsc.load_gather(sig_v, [sidx])
                y = s6 / d6
                seg = bt[pl.ds(v * L, L)]
                plsc.addupdate_scatter(acc, [lane_base + seg], y)
                return 0

            lax.fori_loop(vlo, vhi, v_body, 0, unroll=2)

        pltpu.sync_copy(mp.at[0, pl.ds(tile_base, BLK)], idx_i[0])
        pltpu.sync_copy(mp.at[1, pl.ds(tile_base, BLK)], idx_j[0])
        pltpu.sync_copy(mb.at[pl.ds(tile_base, BLK)], bat[0])

        HA = 992
        HB = BLK - HA

        def blk_body(p, _):
            for b2 in range(2):
                b = 2 * p + b2
                u = b2
                nu = 1 - b2
                ii = idx_i[u]
                jj = idx_j[u]
                g1 = pltpu.async_copy(tab.at[ii.at[pl.ds(0, HA)]],
                                      rows_i.at[pl.ds(0, HA)], sem_g[0])
                g2 = pltpu.async_copy(tab.at[jj.at[pl.ds(0, HA)]],
                                      rows_j.at[pl.ds(0, HA)], sem_g[1])
                g3 = pltpu.async_copy(tab.at[ii.at[pl.ds(HA, HB)]],
                                      rows_i.at[pl.ds(HA, HB)], sem_g[2])
                g4 = pltpu.async_copy(tab.at[jj.at[pl.ds(HA, HB)]],
                                      rows_j.at[pl.ds(HA, HB)], sem_g[3])
                bn = jnp.minimum(b + 1, nblk - 1)
                base = tile_base + bn * BLK
                h1 = pltpu.async_copy(mp.at[0, pl.ds(base, BLK)], idx_i[nu],
                                      sem_i[0])
                h2 = pltpu.async_copy(mp.at[1, pl.ds(base, BLK)], idx_j[nu],
                                      sem_i[1])
                h3 = pltpu.async_copy(mb.at[pl.ds(base, BLK)], bat[nu],
                                      sem_i[2])
                g1.wait()
                g2.wait()
                h1.wait()
                h2.wait()
                h3.wait()
                compute(u, 0, HA // L)
                g3.wait()
                g4.wait()
                compute(u, HA // L, BLK // L)
            return 0

        lax.fori_loop(0, nblk // 2, blk_body, 0)

        for ch in range(NSEG // L):
            tot = zero
            for l in range(NS):
                tot = tot + acc[pl.ds(l * NSEG + ch * L, L)]
            part[pl.ds(ch * L, L)] = tot

        pltpu.sync_copy(part, shared.at[s])
        plsc.subcore_barrier()

        @pl.when(s == 0)
        def _():
            pltpu.sync_copy(shared, red)
            for ch in range(NSEG // L):
                tot2 = jnp.zeros((L,), jnp.float32)
                for l in range(NS):
                    tot2 = tot2 + red[l, pl.ds(ch * L, L)]
                part[pl.ds(ch * L, L)] = tot2
            pltpu.sync_copy(part, out.at[c])

    return k


def kernel(pos, sigma, mapping, mapping_batch, atom_types):
    n_edges = mapping.shape[1]
    node_tab = jnp.concatenate(
        [pos, atom_types.astype(jnp.float32)[:, None]], axis=1)
    s2 = sigma.astype(jnp.float32)
    s2 = s2 * s2
    s6 = (s2 * s2 * s2).reshape(-1)
    sig6 = jnp.pad(s6, (0, 128 - s6.shape[0]))
    mp = mapping.astype(jnp.int32)
    mb = mapping_batch.astype(jnp.int32)
    out = _sc_kernel(n_edges)(node_tab, mp, mb, sig6)
    return out[0] + out[1]


# speedup vs baseline: 1.2161x; 1.2161x over previous
import functools

import jax
import jax.numpy as jnp
from jax import lax
from jax.experimental import pallas as pl
from jax.experimental.pallas import tpu as pltpu
from jax.experimental.pallas import tpu_sc as plsc

NC = 2
NS = 16
L = 16
NW = NC * NS
NSEG = 64
BLK = 2000
H = BLK // 2


def _sc_kernel(n_edges):
    ept = n_edges // NW
    nblk = ept // BLK
    mesh = plsc.VectorSubcoreMesh(core_axis_name="c", subcore_axis_name="s")

    @functools.partial(
        pl.kernel,
        out_type=jax.ShapeDtypeStruct((NC, NSEG), jnp.float32),
        mesh=mesh,
        scratch_types=[
            [pltpu.VMEM((BLK,), jnp.int32) for _ in range(2)],
            [pltpu.VMEM((BLK,), jnp.int32) for _ in range(2)],
            [pltpu.VMEM((BLK,), jnp.int32) for _ in range(2)],
            pltpu.VMEM((BLK, 4), jnp.float32),
            pltpu.VMEM((BLK, 4), jnp.float32),
            pltpu.VMEM((128,), jnp.float32),
            pltpu.VMEM((NS * NSEG,), jnp.float32),
            pltpu.VMEM((NSEG,), jnp.float32),
            pltpu.VMEM((NS, NSEG), jnp.float32),
            pltpu.VMEM_SHARED((NS, NSEG), jnp.float32),
            [pltpu.SemaphoreType.DMA for _ in range(3)],
            [pltpu.SemaphoreType.DMA for _ in range(4)],
        ],
        compiler_params=pltpu.CompilerParams(
            needs_layout_passes=False, use_tc_tiling_on_sc=False),
    )
    def k(tab, mp, mb, sg, out, idx_i, idx_j, bat, rows_i, rows_j, sig_v,
          acc, part, red, shared, sem_i, sem_g):
        c = lax.axis_index("c")
        s = lax.axis_index("s")
        wid = c * NS + s
        tile_base = wid * ept

        pltpu.sync_copy(sg, sig_v)

        zero = jnp.zeros((L,), jnp.float32)

        def zbody(kk, _):
            acc[pl.ds(kk * L, L)] = zero
            return 0

        lax.fori_loop(0, (NS * NSEG) // L, zbody, 0)

        iota = lax.iota(jnp.int32, L)
        lane_base = iota * NSEG
        c0 = jnp.full((L,), 0, jnp.int32)
        c1 = jnp.full((L,), 1, jnp.int32)
        c2 = jnp.full((L,), 2, jnp.int32)
        c3 = jnp.full((L,), 3, jnp.int32)

        def compute(u, vlo, vhi):
            ri = rows_i
            rj = rows_j
            bt = bat[u]

            def v_body(v, _):
                e = v * L + iota
                xi = plsc.load_gather(ri, [e, c0])
                yi = plsc.load_gather(ri, [e, c1])
                zi = plsc.load_gather(ri, [e, c2])
                ti = plsc.load_gather(ri, [e, c3])
                xj = plsc.load_gather(rj, [e, c0])
                yj = plsc.load_gather(rj, [e, c1])
                zj = plsc.load_gather(rj, [e, c2])
                tj = plsc.load_gather(rj, [e, c3])
                dx = xi - xj
                dy = yi - yj
                dz = zi - zj
                d2 = dx * dx + dy * dy + dz * dz + 1e-12
                d6 = d2 * d2 * d2
                sidx = (ti * 10.0 + tj).astype(jnp.int32)
                s6 = plsc.load_gather(sig_v, [sidx])
                y = s6 / d6
                seg = bt[pl.ds(v * L, L)]
                plsc.addupdate_scatter(acc, [lane_base + seg], y)
                return 0

            lax.fori_loop(vlo, vhi, v_body, 0, unroll=2)

        pltpu.sync_copy(mp.at[0, pl.ds(tile_base, BLK)], idx_i[0])
        pltpu.sync_copy(mp.at[1, pl.ds(tile_base, BLK)], idx_j[0])
        pltpu.sync_copy(mb.at[pl.ds(tile_base, BLK)], bat[0])

        CH = (0, 672, 1344, 2000)

        def blk_body(p, _):
            for b2 in range(2):
                b = 2 * p + b2
                u = b2
                nu = 1 - b2
                ii = idx_i[u]
                jj = idx_j[u]

                def gather_chunk(k, sa, sb):
                    lo = CH[k]
                    n = CH[k + 1] - CH[k]
                    gi = pltpu.async_copy(tab.at[ii.at[pl.ds(lo, n)]],
                                          rows_i.at[pl.ds(lo, n)], sa)
                    gj = pltpu.async_copy(tab.at[jj.at[pl.ds(lo, n)]],
                                          rows_j.at[pl.ds(lo, n)], sb)
                    return (gi, gj)

                gA = gather_chunk(0, sem_g[0], sem_g[1])
                bn = jnp.minimum(b + 1, nblk - 1)
                base = tile_base + bn * BLK
                h1 = pltpu.async_copy(mp.at[0, pl.ds(base, BLK)], idx_i[nu],
                                      sem_i[0])
                h2 = pltpu.async_copy(mp.at[1, pl.ds(base, BLK)], idx_j[nu],
                                      sem_i[1])
                h3 = pltpu.async_copy(mb.at[pl.ds(base, BLK)], bat[nu],
                                      sem_i[2])
                gA[0].wait()
                gA[1].wait()
                h1.wait()
                h2.wait()
                h3.wait()
                gB = gather_chunk(1, sem_g[2], sem_g[3])
                compute(u, CH[0] // L, CH[1] // L)
                gB[0].wait()
                gB[1].wait()
                gC = gather_chunk(2, sem_g[0], sem_g[1])
                compute(u, CH[1] // L, CH[2] // L)
                gC[0].wait()
                gC[1].wait()
                compute(u, CH[2] // L, CH[3] // L)
            return 0

        lax.fori_loop(0, nblk // 2, blk_body, 0)

        for ch in range(NSEG // L):
            tot = zero
            for l in range(NS):
                tot = tot + acc[pl.ds(l * NSEG + ch * L, L)]
            part[pl.ds(ch * L, L)] = tot

        pltpu.sync_copy(part, shared.at[s])
        plsc.subcore_barrier()

        @pl.when(s == 0)
        def _():
            pltpu.sync_copy(shared, red)
            for ch in range(NSEG // L):
                tot2 = jnp.zeros((L,), jnp.float32)
                for l in range(NS):
                    tot2 = tot2 + red[l, pl.ds(ch * L, L)]
                part[pl.ds(ch * L, L)] = tot2
            pltpu.sync_copy(part, out.at[c])

    return k


def kernel(pos, sigma, mapping, mapping_batch, atom_types):
    n_edges = mapping.shape[1]
    node_tab = jnp.concatenate(
        [pos, atom_types.astype(jnp.float32)[:, None]], axis=1)
    s2 = sigma.astype(jnp.float32)
    s2 = s2 * s2
    s6 = (s2 * s2 * s2).reshape(-1)
    sig6 = jnp.pad(s6, (0, 128 - s6.shape[0]))
    mp = mapping.astype(jnp.int32)
    mb = mapping_batch.astype(jnp.int32)
    out = _sc_kernel(n_edges)(node_tab, mp, mb, sig6)
    return out[0] + out[1]
